# baseline (speedup 1.0000x reference)
import jax, jax.numpy as jnp
from jax.experimental import pallas as pl

KS3, ST3, PD3, TOPK = 3, 1, 1, 2


def _unfold(x, k, stride, padding):
    B, C, H, W = x.shape
    xp = jnp.pad(x, ((0, 0), (0, 0), (padding, padding), (padding, padding)))
    Hout = (H + 2 * padding - k) // stride + 1
    Wout = (W + 2 * padding - k) // stride + 1
    ii = (jnp.arange(Hout) * stride)[:, None, None, None] + jnp.arange(k)[None, None, :, None]
    jj = (jnp.arange(Wout) * stride)[None, :, None, None] + jnp.arange(k)[None, None, None, :]
    patches = xp[:, :, ii, jj]
    patches = jnp.transpose(patches, (0, 1, 4, 5, 2, 3)).reshape(B, C * k * k, Hout * Wout)
    return patches


def _fold(patches, out_h, out_w, C, k, stride, padding):
    B = patches.shape[0]
    Hout = (out_h + 2 * padding - k) // stride + 1
    Wout = (out_w + 2 * padding - k) // stride + 1
    p = jnp.transpose(patches.reshape(B, C, k, k, Hout, Wout), (0, 1, 4, 5, 2, 3))
    ii = (jnp.arange(Hout) * stride)[:, None, None, None] + jnp.arange(k)[None, None, :, None]
    jj = (jnp.arange(Wout) * stride)[None, :, None, None] + jnp.arange(k)[None, None, None, :]
    out = jnp.zeros((B, C, out_h + 2 * padding, out_w + 2 * padding), patches.dtype)
    out = out.at[:, :, ii, jj].add(p)
    return out[:, :, padding:padding + out_h, padding:padding + out_w]


def _normalize(x, axis):
    n = jnp.linalg.norm(x, axis=axis, keepdims=True)
    return x / jnp.maximum(n, 1e-12)


def _bis(V, idx):
    return jnp.take_along_axis(V, idx[:, None, :], axis=2)


def kernel(lr_lv3, refsr_lv3, ref_lv3, ref_lv2, ref_lv1):
    B, C3, h, w = lr_lv3.shape
    Q = _unfold(lr_lv3, KS3, PD3, ST3)
    K = _unfold(refsr_lv3, KS3, PD3, ST3)
    Kt = _normalize(jnp.transpose(K, (0, 2, 1)), 2)
    Qn = _normalize(Q, 1)
    R = jnp.matmul(Kt, Qn)
    S_vals, H_idx = jax.lax.top_k(jnp.transpose(R, (0, 2, 1)), TOPK)
    soft = jnp.transpose(S_vals, (0, 2, 1))
    hard = jnp.transpose(H_idx, (0, 2, 1))
    C2, C1 = ref_lv2.shape[1], ref_lv1.shape[1]
    V3 = _unfold(ref_lv3, KS3, ST3, PD3)
    V2 = _unfold(ref_lv2, 2 * KS3, 2 * ST3, 2 * PD3)
    V1 = _unfold(ref_lv1, 4 * KS3, 4 * ST3, 4 * PD3)
    S_list, T3_list, T2_list, T1_list = [], [], [], []
    for i in range(TOPK):
        idx = hard[:, i, :]
        t3u = _bis(V3, idx)
        t2u = _bis(V2, idx)
        t1u = _bis(V1, idx)
        d3 = _fold(jnp.ones_like(t3u), h, w, C3, KS3, ST3, PD3)
        d2 = _fold(jnp.ones_like(t2u), 2 * h, 2 * w, C2, 2 * KS3, 2 * ST3, 2 * PD3)
        d1 = _fold(jnp.ones_like(t1u), 4 * h, 4 * w, C1, 4 * KS3, 4 * ST3, 4 * PD3)
        T3 = _fold(t3u, h, w, C3, KS3, ST3, PD3) / d3
        T2 = _fold(t2u, 2 * h, 2 * w, C2, 2 * KS3, 2 * ST3, 2 * PD3) / d2
        T1 = _fold(t1u, 4 * h, 4 * w, C1, 4 * KS3, 4 * ST3, 4 * PD3) / d1
        S_list.append(soft[:, i, :].reshape(B, 1, h // ST3, w // ST3))
        T3_list.append(T3)
        T2_list.append(T2)
        T1_list.append(T1)
    return (jnp.stack(S_list), jnp.stack(T3_list), jnp.stack(T2_list), jnp.stack(T1_list))


# speedup vs baseline: 29.1868x; 29.1868x over previous
import functools

import jax
import jax.numpy as jnp
from jax import lax
from jax.experimental import pallas as pl
from jax.experimental.pallas import tpu as pltpu
from jax.experimental.pallas import tpu_sc as plsc

B = 2
L = 1600
D = 2304
TR = 400
NT = L // TR


def _topk_body(kt_ref, q_ref, soft_ref, hard_ref, m_ref, i_ref):
    t = pl.program_id(1)
    a = kt_ref[0]
    bm = q_ref[0]
    r = lax.dot_general(a, bm, (((1,), (0,)), ((), ())),
                        preferred_element_type=jnp.float32)
    iota = lax.broadcasted_iota(jnp.int32, (TR, L), 0) + t * TR
    big = jnp.int32(1 << 30)
    m1 = jnp.max(r, axis=0)
    i1 = jnp.min(jnp.where(r == m1[None, :], iota, big), axis=0)
    rm = jnp.where(iota == i1[None, :], -jnp.inf, r)
    m2 = jnp.max(rm, axis=0)
    i2 = jnp.min(jnp.where(rm == m2[None, :], iota, big), axis=0)

    @pl.when(t == 0)
    def _():
        m_ref[0], i_ref[0] = m1, i1
        m_ref[1], i_ref[1] = m2, i2

    @pl.when(t > 0)
    def _():
        rm1, ri1 = m_ref[0], i_ref[0]
        rm2, ri2 = m_ref[1], i_ref[1]
        twins = m1 > rm1
        nm1 = jnp.where(twins, m1, rm1)
        ni1 = jnp.where(twins, i1, ri1)
        s_t = m2 > rm1
        s_f = m1 > rm2
        nm2 = jnp.where(twins, jnp.where(s_t, m2, rm1), jnp.where(s_f, m1, rm2))
        ni2 = jnp.where(twins, jnp.where(s_t, i2, ri1), jnp.where(s_f, i1, ri2))
        m_ref[0], i_ref[0] = nm1, ni1
        m_ref[1], i_ref[1] = nm2, ni2

    @pl.when(t == NT - 1)
    def _():
        soft_ref[0] = m_ref[...]
        hard_ref[0] = i_ref[...]


def _stage_a(ktb, qb):
    return pl.pallas_call(
        _topk_body,
        grid=(B, NT),
        in_specs=[
            pl.BlockSpec((1, TR, D), lambda b, t: (b, t, 0)),
            pl.BlockSpec((1, D, L), lambda b, t: (b, 0, 0)),
        ],
        out_specs=[
            pl.BlockSpec((1, 2, L), lambda b, t: (b, 0, 0)),
            pl.BlockSpec((1, 2, L), lambda b, t: (b, 0, 0)),
        ],
        out_shape=[
            jax.ShapeDtypeStruct((B, 2, L), jnp.float32),
            jax.ShapeDtypeStruct((B, 2, L), jnp.int32),
        ],
        scratch_shapes=[
            pltpu.VMEM((2, L), jnp.float32),
            pltpu.VMEM((2, L), jnp.int32),
        ],
        compiler_params=pltpu.CompilerParams(
            dimension_semantics=("arbitrary", "arbitrary")),
    )(ktb, qb)


_LVL = ((1, 40, 1764), (2, 80, 3528), (4, 160, 7056))


def _sc_task(x, j, base, s, tab, out, idxv, idb, gbuf, obuf, sem):
    q = x // s
    r = x - q * s
    edge_q = (q == 0) | (q == 39)
    w3 = jnp.where(edge_q, jnp.float32(1.0 / 6.0), jnp.float32(1.0 / 9.0))
    w2 = jnp.where(edge_q, jnp.float32(1.0 / 4.0), jnp.float32(1.0 / 6.0))
    lane = lax.iota(jnp.int32, 16)
    lane0 = lane * 0
    shift18 = lane0 + 18
    sh = 0
    for di in (-1, 0, 1):
        if di == -1:
            rvec = jnp.full((16,), jnp.where(q > 0, 1, 0), jnp.int32)
        elif di == 1:
            rvec = jnp.full((16,), jnp.where(q < 39, 1, 0), jnp.int32)
        else:
            rvec = None
        u = r + s * (1 - di)
        for dj in (-1, 0, 1):
            off0 = (q + di + 1) * 42 + dj + 1
            svec = jnp.full((16,), u * 42 + base + (1 - dj), jnp.int32)
            for tc in range(3):
                v = idxv[pl.ds(off0 + tc * 16, 16)]
                ri = lax.shift_right_logical(v * 6554, shift18)
                rj = v - ri * 40
                rowid = ri * (42 * s) + rj + svec
                lo = max(0, -dj) - tc * 16
                hi = min(40, 40 - dj) - tc * 16
                if lo > 0:
                    rowid = rowid * jnp.clip(lane - (lo - 1), 0, 1)
                if hi < 16:
                    rowid = rowid * jnp.clip(hi - lane, 0, 1)
                if rvec is not None:
                    rowid = rowid * rvec
                idb[sh, pl.ds(tc * 16, 16)] = rowid
            sh += 1
    cps = [pltpu.async_copy(tab.at[idb.at[shi]], gbuf.at[shi], sem)
           for shi in range(9)]
    for c in cps:
        c.wait()

    def per_cell(cell, car):
        w = jnp.where((cell == 0) | (cell == 39), w2, w3)
        wv = jnp.full((16,), w, jnp.float32)
        for vv in range(16):
            sl = pl.ds(vv * 16, 16)
            acc = gbuf[0, cell, sl]
            for shi in range(1, 9):
                acc = acc + gbuf[shi, cell, sl]
            obuf[pl.ds(cell * 256 + vv * 16, 16)] = acc * wv
        return car

    lax.fori_loop(0, 40, per_cell, 0)
    pltpu.sync_copy(obuf, out.at[j, x])


def _sc_body(idxp, tab3, tab2, tab1, o3, o2, o1, idxv, idb, gbuf, obuf, sem):
    wid = lax.axis_index("s") * 2 + lax.axis_index("c")
    tabs = (tab3, tab2, tab1)
    outs = (o3, o2, o1)

    def per_job(j, car):
        b = lax.rem(j, B)
        pltpu.sync_copy(idxp.at[j], idxv)
        for lv, (s, nx, nrows) in enumerate(_LVL):
            base = 1 + b * nrows

            def per_round(rd, car2, s=s, nx=nx, base=base, lv=lv):
                x = rd * 32 + wid

                @pl.when(x < nx)
                def _():
                    _sc_task(x, j, base, s, tabs[lv], outs[lv],
                             idxv, idb, gbuf, obuf, sem)
                return car2

            lax.fori_loop(0, (nx + 31) // 32, per_round, 0)
        return car

    lax.fori_loop(0, 2 * B, per_job, 0)


def _stage_b(idxp, tab3, tab2, tab1):
    mesh = plsc.VectorSubcoreMesh(core_axis_name="c", subcore_axis_name="s")
    f = pl.kernel(
        _sc_body,
        out_type=[
            jax.ShapeDtypeStruct((2 * B, 40, 10240), jnp.float32),
            jax.ShapeDtypeStruct((2 * B, 80, 10240), jnp.float32),
            jax.ShapeDtypeStruct((2 * B, 160, 10240), jnp.float32),
        ],
        mesh=mesh,
        scratch_types=[
            pltpu.VMEM((1824,), jnp.int32),
            pltpu.VMEM((9, 48), jnp.int32),
            pltpu.VMEM((9, 48, 256), jnp.float32),
            pltpu.VMEM((10240,), jnp.float32),
            pltpu.SemaphoreType.DMA,
        ],
    )
    return f(idxp, tab3, tab2, tab1)


def _normalize(x, axis):
    n = jnp.linalg.norm(x, axis=axis, keepdims=True)
    return x / jnp.maximum(n, 1e-12)


def _unfold3(x):
    Bn, C, H, W = x.shape
    xp = jnp.pad(x, ((0, 0), (0, 0), (1, 1), (1, 1)))
    sl = [xp[:, :, a:a + H, d:d + W] for a in range(3) for d in range(3)]
    return jnp.stack(sl, axis=2).reshape(Bn, C * 9, H * W)


def kernel(lr_lv3, refsr_lv3, ref_lv3, ref_lv2, ref_lv1):
    Q = _unfold3(lr_lv3)
    K = _unfold3(refsr_lv3)
    Kt = _normalize(jnp.transpose(K, (0, 2, 1)), 2)
    Qn = _normalize(Q, 1)
    soft, hard = _stage_a(Kt.astype(jnp.bfloat16), Qn.astype(jnp.bfloat16))

    hardT = jnp.transpose(hard, (1, 0, 2)).reshape(2 * B, 40, 40)
    idxp = jnp.pad(jnp.pad(hardT, ((0, 0), (1, 1), (1, 1))).reshape(2 * B, 1764),
                   ((0, 0), (0, 60)))

    zrow = jnp.zeros((1, 256), jnp.float32)
    p3 = jnp.pad(ref_lv3, ((0, 0), (0, 0), (1, 1), (1, 1)))
    tab3 = jnp.concatenate([zrow, p3.transpose(0, 2, 3, 1).reshape(-1, 256)], 0)
    p2 = jnp.pad(ref_lv2, ((0, 0), (0, 0), (2, 2), (2, 2)))
    tab2 = jnp.concatenate([zrow, p2.transpose(0, 2, 3, 1).reshape(-1, 256)], 0)
    p1 = jnp.pad(ref_lv1, ((0, 0), (0, 0), (4, 4), (4, 4)))
    tab1 = jnp.concatenate([zrow, p1.transpose(0, 2, 3, 1).reshape(-1, 256)], 0)

    o3, o2, o1 = _stage_b(idxp, tab3, tab2, tab1)

    S = jnp.transpose(soft, (1, 0, 2)).reshape(2, B, 1, 40, 40)
    T3 = o3.reshape(2, B, 40, 40, 256).transpose(0, 1, 4, 2, 3)
    T2 = o2.reshape(2, B, 80, 40, 2, 128).transpose(0, 1, 5, 2, 3, 4).reshape(
        2, B, 128, 80, 80)
    T1 = o1.reshape(2, B, 160, 40, 4, 64).transpose(0, 1, 5, 2, 3, 4).reshape(
        2, B, 64, 160, 160)
    return (S, T3, T2, T1)


# speedup vs baseline: 61.7025x; 2.1141x over previous
import functools

import jax
import jax.numpy as jnp
from jax import lax
from jax.experimental import pallas as pl
from jax.experimental.pallas import tpu as pltpu
from jax.experimental.pallas import tpu_sc as plsc

B = 2
L = 1600
D = 2304
TR = 400
NT = L // TR


def _topk_body(kt_ref, q_ref, soft_ref, hard_ref, m_ref, i_ref):
    t = pl.program_id(1)
    a = kt_ref[0]
    bm = q_ref[0]
    r = lax.dot_general(a, bm, (((1,), (0,)), ((), ())),
                        preferred_element_type=jnp.float32)
    iota = lax.broadcasted_iota(jnp.int32, (TR, L), 0) + t * TR
    big = jnp.int32(1 << 30)
    m1 = jnp.max(r, axis=0)
    i1 = jnp.min(jnp.where(r == m1[None, :], iota, big), axis=0)
    rm = jnp.where(iota == i1[None, :], -jnp.inf, r)
    m2 = jnp.max(rm, axis=0)
    i2 = jnp.min(jnp.where(rm == m2[None, :], iota, big), axis=0)

    @pl.when(t == 0)
    def _():
        m_ref[0], i_ref[0] = m1, i1
        m_ref[1], i_ref[1] = m2, i2

    @pl.when(t > 0)
    def _():
        rm1, ri1 = m_ref[0], i_ref[0]
        rm2, ri2 = m_ref[1], i_ref[1]
        twins = m1 > rm1
        nm1 = jnp.where(twins, m1, rm1)
        ni1 = jnp.where(twins, i1, ri1)
        s_t = m2 > rm1
        s_f = m1 > rm2
        nm2 = jnp.where(twins, jnp.where(s_t, m2, rm1), jnp.where(s_f, m1, rm2))
        ni2 = jnp.where(twins, jnp.where(s_t, i2, ri1), jnp.where(s_f, i1, ri2))
        m_ref[0], i_ref[0] = nm1, ni1
        m_ref[1], i_ref[1] = nm2, ni2

    @pl.when(t == NT - 1)
    def _():
        soft_ref[0] = m_ref[...]
        hard_ref[0] = i_ref[...]


def _stage_a(ktb, qb):
    return pl.pallas_call(
        _topk_body,
        grid=(B, NT),
        in_specs=[
            pl.BlockSpec((1, TR, D), lambda b, t: (b, t, 0)),
            pl.BlockSpec((1, D, L), lambda b, t: (b, 0, 0)),
        ],
        out_specs=[
            pl.BlockSpec((1, 2, L), lambda b, t: (b, 0, 0)),
            pl.BlockSpec((1, 2, L), lambda b, t: (b, 0, 0)),
        ],
        out_shape=[
            jax.ShapeDtypeStruct((B, 2, L), jnp.float32),
            jax.ShapeDtypeStruct((B, 2, L), jnp.int32),
        ],
        scratch_shapes=[
            pltpu.VMEM((2, L), jnp.float32),
            pltpu.VMEM((2, L), jnp.int32),
        ],
        compiler_params=pltpu.CompilerParams(
            dimension_semantics=("arbitrary", "arbitrary")),
    )(ktb, qb)


_LVL = ((1, 40, 588), (2, 80, 1176), (4, 160, 2352))


def _sc_task(x, j, base, s, NR, tab, out, idxv, idb, gbuf, obuf, sem):
    q = x // s
    r = x - q * s
    edge_q = (q == 0) | (q == 39)
    w3 = jnp.where(edge_q, jnp.float32(1.0 / 6.0), jnp.float32(1.0 / 9.0))
    w2 = jnp.where(edge_q, jnp.float32(1.0 / 4.0), jnp.float32(1.0 / 6.0))
    lane = lax.iota(jnp.int32, 16)
    lane0 = lane * 0
    shift18 = lane0 + 18
    shift16 = lane0 + 16
    for d, di in enumerate((-1, 0, 1)):
        if di == -1:
            rvec = jnp.full((16,), jnp.where(q > 0, 1, 0), jnp.int32)
        elif di == 1:
            rvec = jnp.full((16,), jnp.where(q < 39, 1, 0), jnp.int32)
        else:
            rvec = None
        u = r + s * (1 - di)
        off0 = (q + di + 1) * 42 + 1
        svec = jnp.full((16,), u * 14 + base, jnp.int32)
        for tc in (2, 0, 1):
            v = idxv[pl.ds(off0 + tc * 16, 16)]
            ri = lax.shift_right_logical(v * 6554, shift18)
            rj = v - ri * 40
            n3 = lax.shift_right_logical(rj * 21846, shift16)
            phi = rj - n3 * 3
            rowid = ri * (14 * s) + n3 + phi * NR + svec
            if rvec is not None:
                rowid = rowid * rvec
            idb[d, pl.ds(tc * 16, 16)] = rowid

    cps = [pltpu.async_copy(tab.at[idb.at[d, pl.ds(0, 40)]],
                            gbuf.at[d, pl.ds(0, 40)], sem)
           for d in range(3)]
    for c in cps:
        c.wait()

    wv3 = jnp.full((16,), w3, jnp.float32)
    wv2 = jnp.full((16,), w2, jnp.float32)

    def per_cell(cell, car):
        for vv in range(16):
            o = vv * 16
            acc = None
            for d in range(3):
                for row_off, col in ((-1, 512), (0, 256), (1, 0)):
                    t = gbuf[d, cell + row_off, pl.ds(col + o, 16)]
                    acc = t if acc is None else acc + t
            obuf[pl.ds(cell * 256 + o, 16)] = acc * wv3
        return car

    lax.fori_loop(1, 39, per_cell, 0)
    for cell, combos in ((0, ((0, 256), (1, 0))), (39, ((-1, 512), (0, 256)))):
        for vv in range(16):
            o = vv * 16
            acc = None
            for d in range(3):
                for row_off, col in combos:
                    t = gbuf[d, cell + row_off, pl.ds(col + o, 16)]
                    acc = t if acc is None else acc + t
            obuf[pl.ds(cell * 256 + o, 16)] = acc * wv2
    pltpu.sync_copy(obuf, out.at[j, x])


def _sc_body(idxp, tab3, tab2, tab1, o3, o2, o1, idxv, idb, gbuf, obuf, sem):
    wid = lax.axis_index("s") * 2 + lax.axis_index("c")
    tabs = (tab3, tab2, tab1)
    outs = (o3, o2, o1)

    def per_job(j, car):
        b = lax.rem(j, B)
        pltpu.sync_copy(idxp.at[j], idxv)
        for lv, (s, nx, NR) in enumerate(_LVL):
            base = 1 + b * 3 * NR

            def per_round(rd, car2, s=s, nx=nx, base=base, NR=NR, lv=lv):
                x = rd * 32 + wid

                @pl.when(x < nx)
                def _():
                    _sc_task(x, j, base, s, NR, tabs[lv], outs[lv],
                             idxv, idb, gbuf, obuf, sem)
                return car2

            lax.fori_loop(0, (nx + 31) // 32, per_round, 0)
        return car

    lax.fori_loop(0, 2 * B, per_job, 0)


def _stage_b(idxp, tab3, tab2, tab1):
    mesh = plsc.VectorSubcoreMesh(core_axis_name="c", subcore_axis_name="s")
    f = pl.kernel(
        _sc_body,
        out_type=[
            jax.ShapeDtypeStruct((2 * B, 40, 10240), jnp.float32),
            jax.ShapeDtypeStruct((2 * B, 80, 10240), jnp.float32),
            jax.ShapeDtypeStruct((2 * B, 160, 10240), jnp.float32),
        ],
        mesh=mesh,
        scratch_types=[
            pltpu.VMEM((1824,), jnp.int32),
            pltpu.VMEM((4, 48), jnp.int32),
            pltpu.VMEM((3, 40, 768), jnp.float32),
            pltpu.VMEM((10240,), jnp.float32),
            pltpu.SemaphoreType.DMA,
        ],
    )
    return f(idxp, tab3, tab2, tab1)


def _normalize(x, axis):
    n = jnp.linalg.norm(x, axis=axis, keepdims=True)
    return x / jnp.maximum(n, 1e-12)


def _mk_tab(refx, s):
    Bn, C, H, W = refx.shape
    p = jnp.pad(refx, ((0, 0), (0, 0), (s, s), (s, s + 2 * s)))
    pcl = p.transpose(0, 2, 3, 1).reshape(Bn, 42 * s, 44 * s * C)
    phases = [pcl[:, :, phi * s * C: (phi + 42) * s * C].reshape(
        Bn, 42 * s * 14, 3 * s * C) for phi in range(3)]
    tabs = jnp.concatenate(phases, axis=1).reshape(-1, 3 * s * C)
    zrow = jnp.zeros((1, 3 * s * C), jnp.float32)
    return jnp.concatenate([zrow, tabs], 0)


def _unfold3(x):
    Bn, C, H, W = x.shape
    xp = jnp.pad(x, ((0, 0), (0, 0), (1, 1), (1, 1)))
    sl = [xp[:, :, a:a + H, d:d + W] for a in range(3) for d in range(3)]
    return jnp.stack(sl, axis=2).reshape(Bn, C * 9, H * W)


def kernel(lr_lv3, refsr_lv3, ref_lv3, ref_lv2, ref_lv1):
    Q = _unfold3(lr_lv3)
    K = _unfold3(refsr_lv3)
    Kt = _normalize(jnp.transpose(K, (0, 2, 1)), 2)
    Qn = _normalize(Q, 1)
    soft, hard = _stage_a(Kt.astype(jnp.bfloat16), Qn.astype(jnp.bfloat16))

    hardT = jnp.transpose(hard, (1, 0, 2)).reshape(2 * B, 40, 40)
    idxp = jnp.pad(jnp.pad(hardT, ((0, 0), (1, 1), (1, 1))).reshape(2 * B, 1764),
                   ((0, 0), (0, 60)))

    tab3 = _mk_tab(ref_lv3, 1)
    tab2 = _mk_tab(ref_lv2, 2)
    tab1 = _mk_tab(ref_lv1, 4)

    o3, o2, o1 = _stage_b(idxp, tab3, tab2, tab1)

    S = jnp.transpose(soft, (1, 0, 2)).reshape(2, B, 1, 40, 40)
    T3 = o3.reshape(2, B, 40, 40, 256).transpose(0, 1, 4, 2, 3)
    T2 = o2.reshape(2, B, 80, 40, 2, 128).transpose(0, 1, 5, 2, 3, 4).reshape(
        2, B, 128, 80, 80)
    T1 = o1.reshape(2, B, 160, 40, 4, 64).transpose(0, 1, 5, 2, 3, 4).reshape(
        2, B, 64, 160, 160)
    return (S, T3, T2, T1)
